# Initial kernel scaffold; baseline (speedup 1.0000x reference)
import jax
import jax.numpy as jnp
from jax.experimental import pallas as pl


def kernel(input, W):
    raise NotImplementedError("write your pallas kernel here")


# speedup vs baseline: 1.8170x; 1.8170x over previous
import functools

import jax
import jax.numpy as jnp
from jax import lax
from jax.experimental import pallas as pl
from jax.experimental.pallas import tpu as pltpu

NUM_EXPERTS = 64
TOPK = 2
TOKENS = 16384
HIDDEN = 2048
BLOCK_T = 1024


def _router_body(x_ref, w_ref, scores_ref, idx_ref):
    logits = lax.dot_general(
        x_ref[...], w_ref[...],
        dimension_numbers=(((1,), (1,)), ((), ())),
        preferred_element_type=jnp.float32,
    )

    eids = lax.broadcasted_iota(jnp.int32, logits.shape, 1)
    m1 = jnp.max(logits, axis=1, keepdims=True)
    i1 = jnp.min(jnp.where(logits == m1, eids, NUM_EXPERTS), axis=1, keepdims=True)
    masked = jnp.where(eids == i1, -jnp.inf, logits)
    m2 = jnp.max(masked, axis=1, keepdims=True)
    i2 = jnp.min(jnp.where(masked == m2, eids, NUM_EXPERTS), axis=1, keepdims=True)

    e2 = jnp.exp(m2 - m1)
    denom = 1.0 + e2
    s1 = 1.0 / denom
    s2 = e2 / denom

    scores_ref[...] = jnp.concatenate([s1, s2], axis=1)
    idx_ref[...] = jnp.concatenate([i1, i2], axis=1)


@jax.jit
def kernel(input, W):
    n_tok = input.shape[0]
    grid = (n_tok // BLOCK_T,)
    scores, indices = pl.pallas_call(
        _router_body,
        grid=grid,
        in_specs=[
            pl.BlockSpec((BLOCK_T, HIDDEN), lambda i: (i, 0)),
            pl.BlockSpec((NUM_EXPERTS, HIDDEN), lambda i: (0, 0)),
        ],
        out_specs=[
            pl.BlockSpec((BLOCK_T, TOPK), lambda i: (i, 0)),
            pl.BlockSpec((BLOCK_T, TOPK), lambda i: (i, 0)),
        ],
        out_shape=[
            jax.ShapeDtypeStruct((n_tok, TOPK), jnp.float32),
            jax.ShapeDtypeStruct((n_tok, TOPK), jnp.int32),
        ],
    )(input, W)
    return scores, indices


# speedup vs baseline: 1.8910x; 1.0407x over previous
import functools

import jax
import jax.numpy as jnp
from jax import lax
from jax.experimental import pallas as pl
from jax.experimental.pallas import tpu as pltpu

NUM_EXPERTS = 64
TOPK = 2
TOKENS = 16384
HIDDEN = 2048
BLOCK_T = 2048


def _router_body(x_ref, w_ref, scores_ref, idx_ref):
    logits = lax.dot_general(
        x_ref[...], w_ref[...],
        dimension_numbers=(((1,), (1,)), ((), ())),
        preferred_element_type=jnp.float32,
    )

    eids = lax.broadcasted_iota(jnp.int32, logits.shape, 1)
    m1 = jnp.max(logits, axis=1, keepdims=True)
    i1 = jnp.min(jnp.where(logits == m1, eids, NUM_EXPERTS), axis=1, keepdims=True)
    masked = jnp.where(eids == i1, -jnp.inf, logits)
    m2 = jnp.max(masked, axis=1, keepdims=True)
    i2 = jnp.min(jnp.where(masked == m2, eids, NUM_EXPERTS), axis=1, keepdims=True)

    e2 = jnp.exp(m2 - m1)
    denom = 1.0 + e2
    s1 = 1.0 / denom
    s2 = e2 / denom

    scores_ref[...] = jnp.concatenate([s1, s2], axis=1)
    idx_ref[...] = jnp.concatenate([i1, i2], axis=1)


@jax.jit
def kernel(input, W):
    n_tok = input.shape[0]
    grid = (n_tok // BLOCK_T,)
    scores, indices = pl.pallas_call(
        _router_body,
        grid=grid,
        in_specs=[
            pl.BlockSpec((BLOCK_T, HIDDEN), lambda i: (i, 0)),
            pl.BlockSpec((NUM_EXPERTS, HIDDEN), lambda i: (0, 0)),
        ],
        out_specs=[
            pl.BlockSpec((BLOCK_T, TOPK), lambda i: (i, 0)),
            pl.BlockSpec((BLOCK_T, TOPK), lambda i: (i, 0)),
        ],
        out_shape=[
            jax.ShapeDtypeStruct((n_tok, TOPK), jnp.float32),
            jax.ShapeDtypeStruct((n_tok, TOPK), jnp.int32),
        ],
    )(input, W)
    return scores, indices
